# baseline (speedup 1.0000x reference)
import functools

import jax
import jax.numpy as jnp
from jax import lax
from jax.experimental import pallas as pl
from jax.experimental.pallas import tpu as pltpu
from jax.experimental.pallas import tpu_sc as plsc

_N = 10000
_E = 320000
_D = 128
_ED = 16
_DH = _D // 2

_NC = 2
_NS = 16
_EP = 327680
_NPAD = 10240
_EPT = _EP // _NS
_CHUNK = 64
_KCH = _EPT // _CHUNK
_BBLK = 20
_NBLK = _KCH // _BBLK
_RPT = _NPAD // _NS
_ZR = 32
_WR = 632
_WT = _N - (_NS - 1) * _WR
_FVH = _DH // 16


def _edge_proj_body(ea_ref, w2_ref, be2_ref, out_ref):
    for cc in range(_NC):
        out_ref[cc] = (
            jnp.dot(ea_ref[...], w2_ref[cc],
                    preferred_element_type=jnp.float32,
                    precision=lax.Precision.HIGHEST)
            + be2_ref[cc])


def _edge_proj(ea2, W2, be2):
    blk = 2048
    grid = (_EP // 2 // blk,)
    return pl.pallas_call(
        _edge_proj_body,
        grid=grid,
        in_specs=[
            pl.BlockSpec((blk, 2 * _ED), lambda i: (i, 0)),
            pl.BlockSpec((_NC, 2 * _ED, _D), lambda i: (0, 0, 0)),
            pl.BlockSpec((_NC, 1, _D), lambda i: (0, 0, 0)),
        ],
        out_specs=pl.BlockSpec((_NC, blk, _D), lambda i: (0, i, 0)),
        out_shape=jax.ShapeDtypeStruct((_NC, _EP // 2, _D), jnp.float32),
    )(ea2, W2, be2)


def _pack_proj_weights(We, be):
    z = jnp.zeros((_ED, _DH), jnp.float32)
    halves = (We[:, :_DH], We[:, _DH:])
    W2 = jnp.stack([
        jnp.concatenate([jnp.concatenate([h, z], axis=1),
                         jnp.concatenate([z, h], axis=1)], axis=0)
        for h in halves])
    be2 = jnp.stack([jnp.concatenate([be[:_DH], be[:_DH]]),
                     jnp.concatenate([be[_DH:], be[_DH:]])]).reshape(_NC, 1, _D)
    return W2, be2


def _node_update_body(x_ref, a_ref, w_ref, b_ref, out_ref, both_ref):
    acc = x_ref[...] + jnp.concatenate(
        [a_ref[0][:, :_DH], a_ref[1][:, :_DH]], axis=1)
    r = (jnp.dot(acc, w_ref[...], preferred_element_type=jnp.float32,
                 precision=lax.Precision.HIGHEST)
         + b_ref[...])
    h = jnp.maximum(r, 0.0)
    out_ref[...] = h
    both_ref[0] = h
    both_ref[1] = jnp.concatenate([h[:, _DH:], h[:, :_DH]], axis=1)


def _node_update(x, parts, W, b):
    blk = 2000
    grid = (_N // blk,)
    return pl.pallas_call(
        _node_update_body,
        grid=grid,
        in_specs=[
            pl.BlockSpec((blk, _D), lambda i: (i, 0)),
            pl.BlockSpec((_NC, blk, _D), lambda i: (0, i, 0)),
            pl.BlockSpec((_D, _D), lambda i: (0, 0)),
            pl.BlockSpec((1, _D), lambda i: (0, 0)),
        ],
        out_specs=[
            pl.BlockSpec((blk, _D), lambda i: (i, 0)),
            pl.BlockSpec((_NC, blk, _D), lambda i: (0, i, 0)),
        ],
        out_shape=[
            jax.ShapeDtypeStruct((_N, _D), jnp.float32),
            jax.ShapeDtypeStruct((_NC, _N, _D), jnp.float32),
        ],
    )(x, parts, W, b.reshape(1, _D))


def _final_body(x_ref, wl_ref, bl_ref, out_ref):
    out_ref[...] = (
        jnp.dot(x_ref[...], wl_ref[...], preferred_element_type=jnp.float32,
                precision=lax.Precision.HIGHEST)
        + bl_ref[...])


def _final(x, Wl, bl):
    blk = 2000
    grid = (_N // blk,)
    return pl.pallas_call(
        _final_body,
        grid=grid,
        in_specs=[
            pl.BlockSpec((blk, _D), lambda i: (i, 0)),
            pl.BlockSpec((_D, _D), lambda i: (0, 0)),
            pl.BlockSpec((1, _D), lambda i: (0, 0)),
        ],
        out_specs=pl.BlockSpec((blk, _D), lambda i: (i, 0)),
        out_shape=jax.ShapeDtypeStruct((_N, _D), jnp.float32),
    )(x, Wl, bl.reshape(1, _D))


def _msgpass_body(x_hbm, e_hbm, src_hbm, dst_hbm, out_hbm,
                  srci_v, dsti_v, e_v, x_v, m_v, z_v, aggr_sh,
                  lsem0, lsem1, isem):
    c = lax.axis_index("c")
    s = lax.axis_index("s")

    def _zrow(i, carry):
        for f in range(_FVH):
            z_v[i, pl.ds(f * 16, 16)] = jnp.zeros((16,), jnp.float32)
        return carry
    lax.fori_loop(0, _ZR, _zrow, 0)

    def _zcopy(j, carry):
        pltpu.sync_copy(z_v, aggr_sh.at[pl.ds(s * _RPT + j * _ZR, _ZR), :])
        return carry
    lax.fori_loop(0, _RPT // _ZR, _zcopy, 0)
    plsc.subcore_barrier()

    def _mzrow(i, carry):
        for f in range(_FVH):
            m_v[i, pl.ds(_DH + f * 16, 16)] = jnp.zeros((16,), jnp.float32)
        return carry
    lax.fori_loop(0, _CHUNK, _mzrow, 0)

    base2 = s * (_EPT // 2)
    ibase = s * _EPT

    @pl.loop(0, _KCH)
    def _chunk_loop(chunk):
        pltpu.sync_copy(
            src_hbm.at[pl.ds(ibase + chunk * _CHUNK, _CHUNK)], srci_v)
        pltpu.sync_copy(
            dst_hbm.at[pl.ds(ibase + chunk * _CHUNK, _CHUNK)], dsti_v)
        pltpu.sync_copy(
            e_hbm.at[c].at[
                pl.ds(base2 + chunk * (_CHUNK // 2), _CHUNK // 2), :],
            e_v)
        pltpu.async_copy(x_hbm.at[c].at[srci_v], x_v, lsem0).wait()

        def _mrow(pr, carry):
            for half in range(2):
                i = 2 * pr + half
                for f in range(_FVH):
                    sl = pl.ds(f * 16, 16)
                    esl = pl.ds(half * _DH + f * 16, 16)
                    m = jnp.maximum(x_v[i, sl] + e_v[pr, esl], 0.0)
                    m_v[i, sl] = m
            return carry
        lax.fori_loop(0, _CHUNK // 2, _mrow, 0)

        pltpu.sync_copy(m_v, aggr_sh.at[dsti_v], add=True)

    plsc.subcore_barrier()
    @pl.when(s < _NS - 1)
    def _writeout_main():
        pltpu.sync_copy(
            aggr_sh.at[pl.ds(s * _WR, _WR), :],
            out_hbm.at[c].at[pl.ds(s * _WR, _WR), :])

    @pl.when(s == _NS - 1)
    def _writeout_tail():
        pltpu.sync_copy(
            aggr_sh.at[pl.ds((_NS - 1) * _WR, _WT), :],
            out_hbm.at[c].at[pl.ds((_NS - 1) * _WR, _WT), :])


def _msgpass(x, e_split, src_r, dst_r):
    mesh = plsc.VectorSubcoreMesh(core_axis_name="c", subcore_axis_name="s")
    kern = pl.kernel(
        _msgpass_body,
        out_type=jax.ShapeDtypeStruct((_NC, _N, _D), jnp.float32),
        mesh=mesh,
        scratch_types=[
            pltpu.VMEM((_CHUNK,), jnp.int32),
            pltpu.VMEM((_CHUNK,), jnp.int32),
            pltpu.VMEM((_CHUNK // 2, _D), jnp.float32),
            pltpu.VMEM((_CHUNK, _D), jnp.float32),
            pltpu.VMEM((_CHUNK, _D), jnp.float32),
            pltpu.VMEM((_ZR, _D), jnp.float32),
            pltpu.VMEM_SHARED((_NPAD, _D), jnp.float32),
            pltpu.SemaphoreType.DMA,
            pltpu.SemaphoreType.DMA,
            pltpu.SemaphoreType.DMA,
        ],
    )
    return kern(x, e_split, src_r, dst_r)


def kernel(x, edge_index, edge_attr,
           We1, be1, W1, b1, We2, be2, W2, b2, We3, be3, W3, b3, Wl, bl):
    npad = _EP - _E
    src_r = jnp.concatenate(
        [edge_index[0], (jnp.arange(npad, dtype=jnp.int32) * 131) % _N])
    dst_r = jnp.concatenate(
        [edge_index[1], _N + jnp.arange(npad, dtype=jnp.int32) % (_NPAD - _N)])
    ea2 = jnp.concatenate(
        [edge_attr.reshape(_E // 2, 2 * _ED),
         jnp.zeros((npad // 2, 2 * _ED), jnp.float32)])
    packed = [_pack_proj_weights(We, be)
              for We, be in ((We1, be1), (We2, be2), (We3, be3))]
    W2s = jnp.stack([wb[0] for wb in packed])
    be2s = jnp.stack([wb[1] for wb in packed])
    Ws = jnp.stack([W1, W2, W3])
    bs = jnp.stack([b1, b2, b3])

    x_both = jnp.stack(
        [x, jnp.concatenate([x[:, _DH:], x[:, :_DH]], axis=1)])

    def _layer(carry, wts):
        h, hb = carry
        W2, be2, W, b = wts
        e = _edge_proj(ea2, W2, be2)
        parts = _msgpass(hb, e, src_r, dst_r)
        return _node_update(h, parts, W, b), None

    (h, _), _ = lax.scan(_layer, (x, x_both), (W2s, be2s, Ws, bs))
    return _final(h, Wl, bl)


# speedup vs baseline: 1.2477x; 1.2477x over previous
import functools

import jax
import jax.numpy as jnp
from jax import lax
from jax.experimental import pallas as pl
from jax.experimental.pallas import tpu as pltpu
from jax.experimental.pallas import tpu_sc as plsc

_N = 10000
_E = 320000
_D = 128
_ED = 16
_DH = _D // 2

_NC = 2
_NS = 16
_EP = 327680
_NPAD = 10240
_EPT = _EP // _NS
_CHUNK = 128
_KCH = _EPT // _CHUNK
_BBLK = 20
_NBLK = _KCH // _BBLK
_RPT = _NPAD // _NS
_ZR = 32
_WR = 632
_WT = _N - (_NS - 1) * _WR
_FVH = _DH // 16


def _edge_proj_body(ea_ref, w2_ref, be2_ref, out_ref):
    for cc in range(_NC):
        out_ref[cc] = (
            jnp.dot(ea_ref[...], w2_ref[cc],
                    preferred_element_type=jnp.float32,
                    precision=lax.Precision.HIGHEST)
            + be2_ref[cc])


def _edge_proj(ea2, W2, be2):
    blk = 2048
    grid = (_EP // 2 // blk,)
    return pl.pallas_call(
        _edge_proj_body,
        grid=grid,
        in_specs=[
            pl.BlockSpec((blk, 2 * _ED), lambda i: (i, 0)),
            pl.BlockSpec((_NC, 2 * _ED, _D), lambda i: (0, 0, 0)),
            pl.BlockSpec((_NC, 1, _D), lambda i: (0, 0, 0)),
        ],
        out_specs=pl.BlockSpec((_NC, blk, _D), lambda i: (0, i, 0)),
        out_shape=jax.ShapeDtypeStruct((_NC, _EP // 2, _D), jnp.float32),
    )(ea2, W2, be2)


def _pack_proj_weights(We, be):
    z = jnp.zeros((_ED, _DH), jnp.float32)
    halves = (We[:, :_DH], We[:, _DH:])
    W2 = jnp.stack([
        jnp.concatenate([jnp.concatenate([h, z], axis=1),
                         jnp.concatenate([z, h], axis=1)], axis=0)
        for h in halves])
    be2 = jnp.stack([jnp.concatenate([be[:_DH], be[:_DH]]),
                     jnp.concatenate([be[_DH:], be[_DH:]])]).reshape(_NC, 1, _D)
    return W2, be2


def _node_update_body(x_ref, a_ref, w_ref, b_ref, out_ref, both_ref):
    acc = x_ref[...] + jnp.concatenate(
        [a_ref[0][:, :_DH], a_ref[1][:, :_DH]], axis=1)
    r = (jnp.dot(acc, w_ref[...], preferred_element_type=jnp.float32,
                 precision=lax.Precision.HIGHEST)
         + b_ref[...])
    h = jnp.maximum(r, 0.0)
    out_ref[...] = h
    both_ref[0] = h
    both_ref[1] = jnp.concatenate([h[:, _DH:], h[:, :_DH]], axis=1)


def _node_update(x, parts, W, b):
    blk = 2000
    grid = (_N // blk,)
    return pl.pallas_call(
        _node_update_body,
        grid=grid,
        in_specs=[
            pl.BlockSpec((blk, _D), lambda i: (i, 0)),
            pl.BlockSpec((_NC, blk, _D), lambda i: (0, i, 0)),
            pl.BlockSpec((_D, _D), lambda i: (0, 0)),
            pl.BlockSpec((1, _D), lambda i: (0, 0)),
        ],
        out_specs=[
            pl.BlockSpec((blk, _D), lambda i: (i, 0)),
            pl.BlockSpec((_NC, blk, _D), lambda i: (0, i, 0)),
        ],
        out_shape=[
            jax.ShapeDtypeStruct((_N, _D), jnp.float32),
            jax.ShapeDtypeStruct((_NC, _N, _D), jnp.float32),
        ],
    )(x, parts, W, b.reshape(1, _D))


def _final_body(x_ref, wl_ref, bl_ref, out_ref):
    out_ref[...] = (
        jnp.dot(x_ref[...], wl_ref[...], preferred_element_type=jnp.float32,
                precision=lax.Precision.HIGHEST)
        + bl_ref[...])


def _final(x, Wl, bl):
    blk = 2000
    grid = (_N // blk,)
    return pl.pallas_call(
        _final_body,
        grid=grid,
        in_specs=[
            pl.BlockSpec((blk, _D), lambda i: (i, 0)),
            pl.BlockSpec((_D, _D), lambda i: (0, 0)),
            pl.BlockSpec((1, _D), lambda i: (0, 0)),
        ],
        out_specs=pl.BlockSpec((blk, _D), lambda i: (i, 0)),
        out_shape=jax.ShapeDtypeStruct((_N, _D), jnp.float32),
    )(x, Wl, bl.reshape(1, _D))


def _msgpass_body(x_hbm, e_hbm, src_hbm, dst_hbm, out_hbm,
                  srci_v, dsti_v, e_v, x_v, m_v, z_v, aggr_sh,
                  lsem0, lsem1, isem):
    c = lax.axis_index("c")
    s = lax.axis_index("s")

    def _zrow(i, carry):
        for f in range(_FVH):
            z_v[i, pl.ds(f * 16, 16)] = jnp.zeros((16,), jnp.float32)
        return carry
    lax.fori_loop(0, _ZR, _zrow, 0)

    def _zcopy(j, carry):
        pltpu.sync_copy(z_v, aggr_sh.at[pl.ds(s * _RPT + j * _ZR, _ZR), :])
        return carry
    lax.fori_loop(0, _RPT // _ZR, _zcopy, 0)
    plsc.subcore_barrier()

    def _mzrow(i, carry):
        for f in range(_FVH):
            m_v[i, pl.ds(_DH + f * 16, 16)] = jnp.zeros((16,), jnp.float32)
        return carry
    lax.fori_loop(0, _CHUNK, _mzrow, 0)

    base2 = s * (_EPT // 2)
    ibase = s * _EPT

    @pl.loop(0, _KCH)
    def _chunk_loop(chunk):
        pltpu.sync_copy(
            src_hbm.at[pl.ds(ibase + chunk * _CHUNK, _CHUNK)], srci_v)
        pltpu.sync_copy(
            dst_hbm.at[pl.ds(ibase + chunk * _CHUNK, _CHUNK)], dsti_v)
        pltpu.sync_copy(
            e_hbm.at[c].at[
                pl.ds(base2 + chunk * (_CHUNK // 2), _CHUNK // 2), :],
            e_v)
        pltpu.async_copy(x_hbm.at[c].at[srci_v], x_v, lsem0).wait()

        def _mrow(pr, carry):
            for half in range(2):
                i = 2 * pr + half
                for f in range(_FVH):
                    sl = pl.ds(f * 16, 16)
                    esl = pl.ds(half * _DH + f * 16, 16)
                    m = jnp.maximum(x_v[i, sl] + e_v[pr, esl], 0.0)
                    m_v[i, sl] = m
            return carry
        lax.fori_loop(0, _CHUNK // 2, _mrow, 0)

        pltpu.sync_copy(m_v, aggr_sh.at[dsti_v], add=True)

    plsc.subcore_barrier()
    @pl.when(s < _NS - 1)
    def _writeout_main():
        pltpu.sync_copy(
            aggr_sh.at[pl.ds(s * _WR, _WR), :],
            out_hbm.at[c].at[pl.ds(s * _WR, _WR), :])

    @pl.when(s == _NS - 1)
    def _writeout_tail():
        pltpu.sync_copy(
            aggr_sh.at[pl.ds((_NS - 1) * _WR, _WT), :],
            out_hbm.at[c].at[pl.ds((_NS - 1) * _WR, _WT), :])


def _msgpass(x, e_split, src_r, dst_r):
    mesh = plsc.VectorSubcoreMesh(core_axis_name="c", subcore_axis_name="s")
    kern = pl.kernel(
        _msgpass_body,
        out_type=jax.ShapeDtypeStruct((_NC, _N, _D), jnp.float32),
        mesh=mesh,
        scratch_types=[
            pltpu.VMEM((_CHUNK,), jnp.int32),
            pltpu.VMEM((_CHUNK,), jnp.int32),
            pltpu.VMEM((_CHUNK // 2, _D), jnp.float32),
            pltpu.VMEM((_CHUNK, _D), jnp.float32),
            pltpu.VMEM((_CHUNK, _D), jnp.float32),
            pltpu.VMEM((_ZR, _D), jnp.float32),
            pltpu.VMEM_SHARED((_NPAD, _D), jnp.float32),
            pltpu.SemaphoreType.DMA,
            pltpu.SemaphoreType.DMA,
            pltpu.SemaphoreType.DMA,
        ],
    )
    return kern(x, e_split, src_r, dst_r)


def kernel(x, edge_index, edge_attr,
           We1, be1, W1, b1, We2, be2, W2, b2, We3, be3, W3, b3, Wl, bl):
    npad = _EP - _E
    src_r = jnp.concatenate(
        [edge_index[0], (jnp.arange(npad, dtype=jnp.int32) * 131) % _N])
    dst_r = jnp.concatenate(
        [edge_index[1], _N + jnp.arange(npad, dtype=jnp.int32) % (_NPAD - _N)])
    ea2 = jnp.concatenate(
        [edge_attr.reshape(_E // 2, 2 * _ED),
         jnp.zeros((npad // 2, 2 * _ED), jnp.float32)])
    packed = [_pack_proj_weights(We, be)
              for We, be in ((We1, be1), (We2, be2), (We3, be3))]
    W2s = jnp.stack([wb[0] for wb in packed])
    be2s = jnp.stack([wb[1] for wb in packed])
    Ws = jnp.stack([W1, W2, W3])
    bs = jnp.stack([b1, b2, b3])

    x_both = jnp.stack(
        [x, jnp.concatenate([x[:, _DH:], x[:, :_DH]], axis=1)])

    def _layer(carry, wts):
        h, hb = carry
        W2, be2, W, b = wts
        e = _edge_proj(ea2, W2, be2)
        parts = _msgpass(hb, e, src_r, dst_r)
        return _node_update(h, parts, W, b), None

    (h, _), _ = lax.scan(_layer, (x, x_both), (W2s, be2s, Ws, bs))
    return _final(h, Wl, bl)
